# Initial kernel scaffold; baseline (speedup 1.0000x reference)
import jax
import jax.numpy as jnp
from jax.experimental import pallas as pl


def kernel(x, edge_index, params):
    raise NotImplementedError("write your pallas kernel here")


# speedup vs baseline: 13.5773x; 13.5773x over previous
import functools

import jax
import jax.numpy as jnp
from jax import lax
from jax.experimental import pallas as pl
from jax.experimental.pallas import tpu as pltpu
from jax.experimental.pallas import tpu_sc as plsc

F32 = jnp.float32
NC, NS = 2, 16
RPD = 128
DMAS = 16
BLK_E = RPD * DMAS
SLW = 8
STG = 1408


def _mesh():
    return plsc.VectorSubcoreMesh(core_axis_name="c", subcore_axis_name="s",
                                  num_cores=NC, num_subcores=NS)


def _make_spmm(n, e, nsl, edge_split, ones_mode=False):
    rows_per_tile = n // NS
    n_wb = rows_per_tile // STG
    nouts = 2 if edge_split else nsl
    epr_t = e // (NC * NS) if edge_split else e // NS
    nblk = epr_t // BLK_E
    assert rows_per_tile % STG == 0 and epr_t % BLK_E == 0

    n_in = 2 + nsl + (1 if edge_split else 0) + (1 if ones_mode else 0)

    def body(*refs):
        src2, dst2 = refs[0], refs[1]
        gs = refs[2:2 + nsl]
        k = 2 + nsl
        zinit = refs[k] if edge_split else None
        k += 1 if edge_split else 0
        ones_blk = refs[k] if ones_mode else None
        k += 1 if ones_mode else 0
        outs = refs[k:k + nouts]
        sidx, didx, rows, stage, acc, sem_g = refs[k + nouts:]
        cid = lax.axis_index("c")
        sid = lax.axis_index("s")
        row0 = pl.multiple_of(sid * rows_per_tile, 8)

        if ones_mode:
            pltpu.sync_copy(ones_blk, rows)

        def run_slice(sl, init_ref, out_ref, ebase):
            def init_i(i, c):
                r = pl.multiple_of(row0 + i * STG, 8)
                pltpu.sync_copy(init_ref.at[pl.ds(r, STG), :], stage)
                pltpu.sync_copy(stage, acc.at[pl.ds(r, STG), :])
                return c
            lax.fori_loop(0, n_wb, init_i, 0)
            plsc.subcore_barrier()

            def blk(b, c):
                eoff = pl.multiple_of((ebase + b * BLK_E) // RPD, 8)
                pltpu.sync_copy(src2.at[pl.ds(eoff, DMAS), :], sidx)
                pltpu.sync_copy(dst2.at[pl.ds(eoff, DMAS), :], didx)
                if not ones_mode:
                    descs = [pltpu.async_copy(gs[sl].at[sidx.at[j]], rows.at[j],
                                              sem_g) for j in range(DMAS)]
                    for d in descs:
                        d.wait()
                for j in range(DMAS):
                    pltpu.sync_copy(rows.at[j], acc.at[didx.at[j]], add=True)
                return c
            lax.fori_loop(0, nblk, blk, 0)
            plsc.subcore_barrier()

            def wb_i(i, c):
                r = pl.multiple_of(row0 + i * STG, 8)
                pltpu.sync_copy(acc.at[pl.ds(r, STG), :], stage)
                pltpu.sync_copy(stage, out_ref.at[pl.ds(r, STG), :])
                return c
            lax.fori_loop(0, n_wb, wb_i, 0)
            plsc.subcore_barrier()

        if edge_split:
            ebase = (cid * NS + sid) * epr_t

            @pl.when(cid == 0)
            def _():
                run_slice(0, gs[0], outs[0], ebase)

            @pl.when(cid == 1)
            def _():
                run_slice(0, zinit, outs[1], ebase)
        else:
            for sl in range(nsl):
                @pl.when(cid == (sl % NC))
                def _(sl=sl):
                    run_slice(sl, gs[sl], outs[sl], sid * epr_t)

    out_type = [jax.ShapeDtypeStruct((n, SLW), F32)] * nouts
    scratch = [
        pltpu.VMEM((DMAS, RPD), jnp.int32),
        pltpu.VMEM((DMAS, RPD), jnp.int32),
        pltpu.VMEM((DMAS, RPD, SLW), F32),
        pltpu.VMEM((STG, SLW), F32),
        pltpu.VMEM_SHARED((n, SLW), F32),
        pltpu.SemaphoreType.DMA,
    ]
    return pl.kernel(body, out_type=out_type, mesh=_mesh(),
                     scratch_types=scratch,
                     compiler_params=pltpu.CompilerParams(
                         use_tc_tiling_on_sc=False))


_NBLK = 2048


def _full(shape):
    return pl.BlockSpec(shape, lambda i: tuple(0 for _ in shape))


def _rows(w):
    return pl.BlockSpec((_NBLK, w), lambda i: (i, 0))


def _tc_pre0(x, p0, p1, Wg0):
    n = x.shape[0]

    def body(x_r, p0_r, p1_r, w_r, dis_r, *g_r):
        deg = p0_r[:, 0:1] + p1_r[:, 0:1]
        dis = lax.rsqrt(deg)
        dis_r[...] = dis
        g = dis * jnp.dot(x_r[...], w_r[...], preferred_element_type=F32)
        for s in range(4):
            g_r[s][...] = g[:, s * SLW:(s + 1) * SLW]

    outs = ([jax.ShapeDtypeStruct((n, 1), F32)] +
            [jax.ShapeDtypeStruct((n, SLW), F32)] * 4)
    return pl.pallas_call(
        body, grid=(n // _NBLK,),
        in_specs=[_rows(3), _rows(SLW), _rows(SLW), _full((3, 32))],
        out_specs=[_rows(1)] + [_rows(SLW)] * 4,
        out_shape=outs)(x, p0, p1, Wg0)


def _tc_fuse(s_parts, dis, b, Wn, nsl_out, relu=True):
    n = dis.shape[0]
    nin = len(s_parts)
    wi, wo = Wn.shape

    def body(*refs):
        s_r = refs[:nin]
        dis_r, b_r, w_r = refs[nin], refs[nin + 1], refs[nin + 2]
        g_r = refs[nin + 3:]
        sg = jnp.concatenate([r[...] for r in s_r], axis=1)
        dis = dis_r[...]
        agg = dis * sg + b_r[...]
        if relu:
            agg = jnp.maximum(agg, 0.0)
        g = dis * jnp.dot(agg, w_r[...], preferred_element_type=F32)
        for s in range(nsl_out):
            g_r[s][...] = g[:, s * SLW:(s + 1) * SLW]

    return pl.pallas_call(
        body, grid=(n // _NBLK,),
        in_specs=[_rows(SLW)] * nin + [_rows(1), _full((1, wi)), _full((wi, wo))],
        out_specs=[_rows(SLW)] * nsl_out,
        out_shape=[jax.ShapeDtypeStruct((n, SLW), F32)] * nsl_out,
    )(*s_parts, dis, b, Wn)


def _tc_post(s_parts, dis, b, wout, relu):
    n = dis.shape[0]
    nin = len(s_parts)
    wfull = s_parts[0].shape[1] * (nin if nin * SLW == s_parts[0].shape[1] * nin else 1)

    def body(*refs):
        s_r = refs[:nin]
        dis_r, b_r, o_r = refs[nin], refs[nin + 1], refs[nin + 2]
        sg = jnp.concatenate([r[...] for r in s_r], axis=1)
        out = dis_r[...] * sg + b_r[...]
        if relu:
            out = jnp.maximum(out, 0.0)
        o_r[...] = out[:, :wout]

    wcat = nin * SLW
    return pl.pallas_call(
        body, grid=(n // _NBLK,),
        in_specs=[_rows(SLW)] * nin + [_rows(1), _full((1, wcat))],
        out_specs=_rows(wout),
        out_shape=jax.ShapeDtypeStruct((n, wout), F32),
    )(*s_parts, dis, b)


def _tc_post_sum2(p0, p1, g, dis, b, wout):
    n = dis.shape[0]

    def body(p0_r, p1_r, dis_r, b_r, o_r):
        sg = p0_r[...] + p1_r[...]
        o_r[...] = (dis_r[...] * sg + b_r[...])[:, :wout]

    return pl.pallas_call(
        body, grid=(n // _NBLK,),
        in_specs=[_rows(SLW), _rows(SLW), _rows(1), _full((1, SLW))],
        out_specs=_rows(wout),
        out_shape=jax.ShapeDtypeStruct((n, wout), F32),
    )(p0, p1, dis, b)


def _tc_pre_dec(dmat, dis, Wgd8):
    n = dmat.shape[0]

    def body(d_r, dis_r, w_r, g_r):
        g_r[...] = dis_r[...] * jnp.dot(d_r[...], w_r[...],
                                        preferred_element_type=F32)

    return pl.pallas_call(
        body, grid=(n // _NBLK,),
        in_specs=[_rows(32), _rows(1), _full((32, SLW))],
        out_specs=_rows(SLW),
        out_shape=jax.ShapeDtypeStruct((n, SLW), F32),
    )(dmat, dis, Wgd8)


_MBLK = 1024


def _tc_mlp(hf, eps, enc_w, enc_b, Wmu, bmu, Wlv, blv, dec_w, dec_b, Wout, bout):
    m, fin = hf.shape
    lat = Wmu.shape[1]
    ne, nd = len(enc_w), len(dec_w)

    def body(*refs):
        i = 0
        hf_r = refs[i]; i += 1
        eps_r = refs[i]; i += 1
        ew = refs[i:i + ne]; i += ne
        eb = refs[i:i + ne]; i += ne
        wmu_r, bmu_r, wlv_r, blv_r = refs[i:i + 4]; i += 4
        dw = refs[i:i + nd]; i += nd
        db = refs[i:i + nd]; i += nd
        wout_r, bout_r = refs[i:i + 2]; i += 2
        mu_r, lv_r, z_r, d_r = refs[i:i + 4]

        h = hf_r[...]
        for j in range(ne):
            h = jnp.maximum(jnp.dot(h, ew[j][...],
                                    preferred_element_type=F32) + eb[j][...], 0.0)
        mu = jnp.dot(h, wmu_r[...], preferred_element_type=F32) + bmu_r[...]
        lv = jnp.dot(h, wlv_r[...], preferred_element_type=F32) + blv_r[...]
        z = mu + eps_r[...] * jnp.exp(0.5 * lv)
        mu_r[...] = mu
        lv_r[...] = lv
        z_r[...] = z
        d = z
        for j in range(nd):
            d = jnp.maximum(jnp.dot(d, dw[j][...],
                                    preferred_element_type=F32) + db[j][...], 0.0)
        d_r[...] = jnp.dot(d, wout_r[...], preferred_element_type=F32) + bout_r[...]

    row = lambda w: pl.BlockSpec((_MBLK, w), lambda i: (i, 0))
    in_specs = ([row(fin), row(lat)] +
                [_full(w.shape) for w in enc_w] + [_full(b.shape) for b in enc_b] +
                [_full(Wmu.shape), _full(bmu.shape), _full(Wlv.shape), _full(blv.shape)] +
                [_full(w.shape) for w in dec_w] + [_full(b.shape) for b in dec_b] +
                [_full(Wout.shape), _full(bout.shape)])
    outs = [jax.ShapeDtypeStruct((m, lat), F32)] * 3 + \
           [jax.ShapeDtypeStruct((m, fin), F32)]
    return pl.pallas_call(
        body, grid=(m // _MBLK,),
        in_specs=in_specs,
        out_specs=[row(lat)] * 3 + [row(fin)],
        out_shape=outs,
    )(hf, eps, *enc_w, *enc_b, Wmu, bmu, Wlv, blv, *dec_w, *dec_b, Wout, bout)


def kernel(x, edge_index, params):
    n = x.shape[0]
    e = edge_index.shape[1]
    src2 = edge_index[0].reshape(-1, RPD)
    dst2 = edge_index[1].reshape(-1, RPD)

    ones_n = jnp.ones((n, SLW), F32)
    zeros_n = jnp.zeros((n, SLW), F32)
    ones_blk = jnp.ones((DMAS, RPD, SLW), F32)

    deg_k = _make_spmm(n, e, 1, edge_split=True, ones_mode=True)
    p0, p1 = deg_k(src2, dst2, ones_n, zeros_n, ones_blk)

    dis, g00, g01, g02, g03 = _tc_pre0(x, p0, p1, params['Wg0'])

    spmm4 = _make_spmm(n, e, 4, edge_split=False)
    spmm8 = _make_spmm(n, e, 8, edge_split=False)
    spmm1 = _make_spmm(n, e, 1, edge_split=True)

    s0 = spmm4(src2, dst2, g00, g01, g02, g03)
    g1 = _tc_fuse(s0, dis, params['bg0'].reshape(1, -1), params['Wg1'], 8)
    s1 = spmm8(src2, dst2, *g1)
    g2 = _tc_fuse(s1, dis, params['bg1'].reshape(1, -1), params['Wg2'], 4)
    s2 = spmm4(src2, dst2, *g2)
    hmat = _tc_post(s2, dis, params['bg2'].reshape(1, -1), 32, relu=False)

    hf = hmat.reshape(-1, 32 * 22)
    eps = jax.random.normal(jax.random.key(42), (hf.shape[0], 56), F32)
    enc_w = [params['We%d' % i] for i in range(5)]
    enc_b = [params['be%d' % i].reshape(1, -1) for i in range(5)]
    dec_w = [params['Wd%d' % i] for i in range(5)]
    dec_b = [params['bd%d' % i].reshape(1, -1) for i in range(5)]
    mu, logvar, z, dflat = _tc_mlp(
        hf, eps, enc_w, enc_b,
        params['Wmu'], params['bmu'].reshape(1, -1),
        params['Wlv'], params['blv'].reshape(1, -1),
        dec_w, dec_b, params['Wout'], params['bout'].reshape(1, -1))

    dmat = dflat.reshape(-1, 32)
    Wgd8 = jnp.pad(params['Wgd'], ((0, 0), (0, SLW - 3)))
    bgd8 = jnp.pad(params['bgd'], (0, SLW - 3)).reshape(1, -1)
    g3 = _tc_pre_dec(dmat, dis, Wgd8)
    q0, q1 = spmm1(src2, dst2, g3, zeros_n)
    x_rec = _tc_post_sum2(q0, q1, g3, dis, bgd8, 3)

    return (x_rec, z, mu, logvar)


# speedup vs baseline: 20.9841x; 1.5455x over previous
import functools

import jax
import jax.numpy as jnp
from jax import lax
from jax.experimental import pallas as pl
from jax.experimental.pallas import tpu as pltpu
from jax.experimental.pallas import tpu_sc as plsc

F32 = jnp.float32
NC, NS = 2, 16
RPD = 128
DMAS = 16
BLK_E = RPD * DMAS
SLW = 8
STG = 1408


def _mesh():
    return plsc.VectorSubcoreMesh(core_axis_name="c", subcore_axis_name="s",
                                  num_cores=NC, num_subcores=NS)


def _make_spmm(n, e, nsl, edge_split, ones_mode=False):
    rows_per_tile = n // NS
    n_wb = rows_per_tile // STG
    nouts = 2 if edge_split else nsl
    epr_t = e // (NC * NS) if edge_split else e // NS
    nblk = epr_t // BLK_E
    assert rows_per_tile % STG == 0 and epr_t % BLK_E == 0

    n_in = 2 + nsl + (1 if edge_split else 0) + (1 if ones_mode else 0)

    def body(*refs):
        src2, dst2 = refs[0], refs[1]
        gs = refs[2:2 + nsl]
        k = 2 + nsl
        zinit = refs[k] if edge_split else None
        k += 1 if edge_split else 0
        ones_blk = refs[k] if ones_mode else None
        k += 1 if ones_mode else 0
        outs = refs[k:k + nouts]
        sidx, didx, rows, stage, acc, sem_g = refs[k + nouts:]
        cid = lax.axis_index("c")
        sid = lax.axis_index("s")
        row0 = pl.multiple_of(sid * rows_per_tile, 8)

        if ones_mode:
            pltpu.sync_copy(ones_blk, rows)

        def run_slice(sl, init_ref, out_ref, ebase):
            def init_i(i, c):
                r = pl.multiple_of(row0 + i * STG, 8)
                pltpu.sync_copy(init_ref.at[pl.ds(r, STG), :], stage)
                pltpu.sync_copy(stage, acc.at[pl.ds(r, STG), :])
                return c
            lax.fori_loop(0, n_wb, init_i, 0)
            plsc.subcore_barrier()

            def blk(b, c):
                eoff = pl.multiple_of((ebase + b * BLK_E) // RPD, 8)
                pltpu.sync_copy(src2.at[pl.ds(eoff, DMAS), :], sidx)
                pltpu.sync_copy(dst2.at[pl.ds(eoff, DMAS), :], didx)
                if not ones_mode:
                    descs = [pltpu.async_copy(gs[sl].at[sidx.at[j]], rows.at[j],
                                              sem_g) for j in range(DMAS)]
                    for d in descs:
                        d.wait()
                for j in range(DMAS):
                    pltpu.sync_copy(rows.at[j], acc.at[didx.at[j]], add=True)
                return c
            lax.fori_loop(0, nblk, blk, 0)
            plsc.subcore_barrier()

            def wb_i(i, c):
                r = pl.multiple_of(row0 + i * STG, 8)
                pltpu.sync_copy(acc.at[pl.ds(r, STG), :], stage)
                pltpu.sync_copy(stage, out_ref.at[pl.ds(r, STG), :])
                return c
            lax.fori_loop(0, n_wb, wb_i, 0)
            plsc.subcore_barrier()

        if edge_split:
            ebase = (cid * NS + sid) * epr_t

            @pl.when(cid == 0)
            def _():
                run_slice(0, gs[0], outs[0], ebase)

            @pl.when(cid == 1)
            def _():
                run_slice(0, zinit, outs[1], ebase)
        else:
            for sl in range(nsl):
                @pl.when(cid == (sl % NC))
                def _(sl=sl):
                    run_slice(sl, gs[sl], outs[sl], sid * epr_t)

    out_type = [jax.ShapeDtypeStruct((n, SLW), F32)] * nouts
    scratch = [
        pltpu.VMEM((DMAS, RPD), jnp.int32),
        pltpu.VMEM((DMAS, RPD), jnp.int32),
        pltpu.VMEM((DMAS, RPD, SLW), F32),
        pltpu.VMEM((STG, SLW), F32),
        pltpu.VMEM_SHARED((n, SLW), F32),
        pltpu.SemaphoreType.DMA,
    ]
    return pl.kernel(body, out_type=out_type, mesh=_mesh(),
                     scratch_types=scratch,
                     compiler_params=pltpu.CompilerParams(
                         use_tc_tiling_on_sc=False))


_NBLK = 2048
_PBLK = _NBLK // 16


def _full(shape):
    return pl.BlockSpec(shape, lambda i: tuple(0 for _ in shape))


def _rows(w):
    return pl.BlockSpec((_NBLK, w), lambda i: (i, 0))


def _prow():
    return pl.BlockSpec((_PBLK, 128), lambda i: (i, 0))


def _pout(n):
    return jax.ShapeDtypeStruct((n // 16, 128), F32)


def _tc_pre0(xP, p0P, p1P, W8):
    n16 = xP.shape[0]

    def body(x_r, p0_r, p1_r, w_r, disP_r, *g_r):
        disP = lax.rsqrt(p0_r[...] + p1_r[...])
        disP_r[...] = disP
        gk = []
        for k in range(16):
            xk = x_r[:, 8 * k:8 * k + 8]
            d = disP[:, 8 * k:8 * k + 1]
            gk.append(d * jnp.dot(xk, w_r[...], preferred_element_type=F32))
        for s in range(4):
            g_r[s][...] = jnp.concatenate(
                [gk[k][:, 8 * s:8 * s + 8] for k in range(16)], axis=1)

    outs = [jax.ShapeDtypeStruct((n16, 128), F32)] * 5
    return pl.pallas_call(
        body, grid=(n16 // _PBLK,),
        in_specs=[_prow(), _prow(), _prow(), _full((SLW, 32))],
        out_specs=[_prow()] * 5,
        out_shape=outs)(xP, p0P, p1P, W8)


def _tc_fuse(s_parts, disP, bP, Wn, nsl_out, relu=True):
    nin = len(s_parts)
    wi, wo = Wn.shape
    n16 = disP.shape[0]

    def body(*refs):
        s_r = refs[:nin]
        disP_r = refs[nin]
        b_r = refs[nin + 1:nin + 1 + nin]
        w_r = refs[nin + 1 + nin]
        g_r = refs[nin + 2 + nin:]
        disP = disP_r[...]
        aggs = []
        for s in range(nin):
            a = disP * s_r[s][...] + b_r[s][...]
            if relu:
                a = jnp.maximum(a, 0.0)
            aggs.append(a)
        Gk = []
        for k in range(16):
            Ak = jnp.concatenate([aggs[s][:, 8 * k:8 * k + 8]
                                  for s in range(nin)], axis=1)
            Rk = jnp.dot(Ak, w_r[...], preferred_element_type=F32)
            Gk.append(disP[:, 8 * k:8 * k + 1] * Rk)
        for s2 in range(nsl_out):
            g_r[s2][...] = jnp.concatenate(
                [Gk[k][:, 8 * s2:8 * s2 + 8] for k in range(16)], axis=1)

    return pl.pallas_call(
        body, grid=(n16 // _PBLK,),
        in_specs=([_prow()] * (nin + 1) + [_full((1, 128))] * nin +
                  [_full((wi, wo))]),
        out_specs=[_prow()] * nsl_out,
        out_shape=[jax.ShapeDtypeStruct((n16, 128), F32)] * nsl_out,
    )(*s_parts, disP, *bP, Wn)


def _tc_post(s_parts, p0, p1, b, wout, relu):
    n = p0.shape[0]
    nin = len(s_parts)

    def body(*refs):
        s_r = refs[:nin]
        p0_r, p1_r, b_r, o_r = refs[nin:nin + 4]
        dis = lax.rsqrt(p0_r[:, 0:1] + p1_r[:, 0:1])
        sg = jnp.concatenate([r[...] for r in s_r], axis=1)
        out = dis * sg + b_r[...]
        if relu:
            out = jnp.maximum(out, 0.0)
        o_r[...] = out[:, :wout]

    wcat = nin * SLW
    return pl.pallas_call(
        body, grid=(n // _NBLK,),
        in_specs=[_rows(SLW)] * (nin + 2) + [_full((1, wcat))],
        out_specs=_rows(wout),
        out_shape=jax.ShapeDtypeStruct((n, wout), F32),
    )(*s_parts, p0, p1, b)


def _tc_post_sum2(q0, q1, p0, p1, b, wout):
    n = p0.shape[0]

    def body(q0_r, q1_r, p0_r, p1_r, b_r, o_r):
        dis = lax.rsqrt(p0_r[:, 0:1] + p1_r[:, 0:1])
        sg = q0_r[...] + q1_r[...]
        o_r[...] = (dis * sg + b_r[...])[:, :wout]

    return pl.pallas_call(
        body, grid=(n // _NBLK,),
        in_specs=[_rows(SLW)] * 4 + [_full((1, SLW))],
        out_specs=_rows(wout),
        out_shape=jax.ShapeDtypeStruct((n, wout), F32),
    )(q0, q1, p0, p1, b)


def _tc_pre_dec(dmat, p0, p1, Wgd8):
    n = dmat.shape[0]

    def body(d_r, p0_r, p1_r, w_r, g_r):
        dis = lax.rsqrt(p0_r[:, 0:1] + p1_r[:, 0:1])
        g_r[...] = dis * jnp.dot(d_r[...], w_r[...],
                                 preferred_element_type=F32)

    return pl.pallas_call(
        body, grid=(n // _NBLK,),
        in_specs=[_rows(32), _rows(SLW), _rows(SLW), _full((32, SLW))],
        out_specs=_rows(SLW),
        out_shape=jax.ShapeDtypeStruct((n, SLW), F32),
    )(dmat, p0, p1, Wgd8)


_MBLK = 1024


def _tc_mlp(hf, eps, enc_w, enc_b, Wmu, bmu, Wlv, blv, dec_w, dec_b, Wout, bout):
    m, fin = hf.shape
    lat = Wmu.shape[1]
    ne, nd = len(enc_w), len(dec_w)

    def body(*refs):
        i = 0
        hf_r = refs[i]; i += 1
        eps_r = refs[i]; i += 1
        ew = refs[i:i + ne]; i += ne
        eb = refs[i:i + ne]; i += ne
        wmu_r, bmu_r, wlv_r, blv_r = refs[i:i + 4]; i += 4
        dw = refs[i:i + nd]; i += nd
        db = refs[i:i + nd]; i += nd
        wout_r, bout_r = refs[i:i + 2]; i += 2
        mu_r, lv_r, z_r, d_r = refs[i:i + 4]

        h = hf_r[...]
        for j in range(ne):
            h = jnp.maximum(jnp.dot(h, ew[j][...],
                                    preferred_element_type=F32) + eb[j][...], 0.0)
        mu = jnp.dot(h, wmu_r[...], preferred_element_type=F32) + bmu_r[...]
        lv = jnp.dot(h, wlv_r[...], preferred_element_type=F32) + blv_r[...]
        z = mu + eps_r[...] * jnp.exp(0.5 * lv)
        mu_r[...] = mu
        lv_r[...] = lv
        z_r[...] = z
        d = z
        for j in range(nd):
            d = jnp.maximum(jnp.dot(d, dw[j][...],
                                    preferred_element_type=F32) + db[j][...], 0.0)
        d_r[...] = jnp.dot(d, wout_r[...], preferred_element_type=F32) + bout_r[...]

    row = lambda w: pl.BlockSpec((_MBLK, w), lambda i: (i, 0))
    in_specs = ([row(fin), row(lat)] +
                [_full(w.shape) for w in enc_w] + [_full(b.shape) for b in enc_b] +
                [_full(Wmu.shape), _full(bmu.shape), _full(Wlv.shape), _full(blv.shape)] +
                [_full(w.shape) for w in dec_w] + [_full(b.shape) for b in dec_b] +
                [_full(Wout.shape), _full(bout.shape)])
    outs = [jax.ShapeDtypeStruct((m, lat), F32)] * 3 + \
           [jax.ShapeDtypeStruct((m, fin), F32)]
    return pl.pallas_call(
        body, grid=(m // _MBLK,),
        in_specs=in_specs,
        out_specs=[row(lat)] * 3 + [row(fin)],
        out_shape=outs,
    )(hf, eps, *enc_w, *enc_b, Wmu, bmu, Wlv, blv, *dec_w, *dec_b, Wout, bout)


def kernel(x, edge_index, params):
    n = x.shape[0]
    e = edge_index.shape[1]
    src2 = edge_index[0].reshape(-1, RPD)
    dst2 = edge_index[1].reshape(-1, RPD)

    sc = lambda a: a.reshape(n, SLW)
    tc = lambda a: a.reshape(n // 16, 128)
    bP = lambda b, nsl: [jnp.tile(b[SLW * s:SLW * (s + 1)], 16).reshape(1, 128)
                         for s in range(nsl)]

    ones_n = jnp.ones((n, SLW), F32)
    zeros_n = jnp.zeros((n, SLW), F32)
    ones_blk = jnp.ones((DMAS, RPD, SLW), F32)

    deg_k = _make_spmm(n, e, 1, edge_split=True, ones_mode=True)
    p0, p1 = deg_k(src2, dst2, ones_n, zeros_n, ones_blk)

    xP = tc(jnp.pad(x, ((0, 0), (0, SLW - 3))))
    W8 = jnp.pad(params['Wg0'], ((0, SLW - 3), (0, 0)))
    disP, g00, g01, g02, g03 = _tc_pre0(xP, tc(p0), tc(p1), W8)

    spmm4 = _make_spmm(n, e, 4, edge_split=False)
    spmm8 = _make_spmm(n, e, 8, edge_split=False)
    spmm1 = _make_spmm(n, e, 1, edge_split=True)

    s0 = spmm4(src2, dst2, sc(g00), sc(g01), sc(g02), sc(g03))
    g1 = _tc_fuse([tc(a) for a in s0], disP, bP(params['bg0'], 4),
                  params['Wg1'], 8)
    s1 = spmm8(src2, dst2, *[sc(a) for a in g1])
    g2 = _tc_fuse([tc(a) for a in s1], disP, bP(params['bg1'], 8),
                  params['Wg2'], 4)
    s2 = spmm4(src2, dst2, *[sc(a) for a in g2])
    hmat = _tc_post(s2, p0, p1, params['bg2'].reshape(1, -1), 32, relu=False)

    hf = hmat.reshape(-1, 32 * 22)
    eps = jax.random.normal(jax.random.key(42), (hf.shape[0], 56), F32)
    enc_w = [params['We%d' % i] for i in range(5)]
    enc_b = [params['be%d' % i].reshape(1, -1) for i in range(5)]
    dec_w = [params['Wd%d' % i] for i in range(5)]
    dec_b = [params['bd%d' % i].reshape(1, -1) for i in range(5)]
    mu, logvar, z, dflat = _tc_mlp(
        hf, eps, enc_w, enc_b,
        params['Wmu'], params['bmu'].reshape(1, -1),
        params['Wlv'], params['blv'].reshape(1, -1),
        dec_w, dec_b, params['Wout'], params['bout'].reshape(1, -1))

    dmat = dflat.reshape(-1, 32)
    Wgd8 = jnp.pad(params['Wgd'], ((0, 0), (0, SLW - 3)))
    bgd8 = jnp.pad(params['bgd'], (0, SLW - 3)).reshape(1, -1)
    g3 = _tc_pre_dec(dmat, p0, p1, Wgd8)
    q0, q1 = spmm1(src2, dst2, g3, zeros_n)
    x_rec = _tc_post_sum2(q0, q1, p0, p1, bgd8, 3)

    return (x_rec, z, mu, logvar)


# speedup vs baseline: 24.7650x; 1.1802x over previous
import functools

import jax
import jax.numpy as jnp
from jax import lax
from jax.experimental import pallas as pl
from jax.experimental.pallas import tpu as pltpu
from jax.experimental.pallas import tpu_sc as plsc

F32 = jnp.float32
NC, NS = 2, 16
RPD = 128
DMAS = 16
BLK_E = RPD * DMAS
SLW = 8
STG = 1408


def _mesh():
    return plsc.VectorSubcoreMesh(core_axis_name="c", subcore_axis_name="s",
                                  num_cores=NC, num_subcores=NS)


def _make_spmm(n, e, nsl, edge_split, ones_mode=False):
    rows_per_tile = n // NS
    n_wb = rows_per_tile // STG
    nouts = 2 if edge_split else nsl
    epr_t = e // (NC * NS) if edge_split else e // NS
    nblk = epr_t // BLK_E
    assert rows_per_tile % STG == 0 and epr_t % BLK_E == 0

    n_in = 2 + nsl + (1 if edge_split else 0) + (1 if ones_mode else 0)

    def body(*refs):
        src2, dst2 = refs[0], refs[1]
        gs = refs[2:2 + nsl]
        k = 2 + nsl
        zinit = refs[k] if edge_split else None
        k += 1 if edge_split else 0
        ones_blk = refs[k] if ones_mode else None
        k += 1 if ones_mode else 0
        outs = refs[k:k + nouts]
        sidx, didx, rows, stage, acc, sem_g = refs[k + nouts:]
        cid = lax.axis_index("c")
        sid = lax.axis_index("s")
        row0 = pl.multiple_of(sid * rows_per_tile, 8)

        if ones_mode:
            pltpu.sync_copy(ones_blk, rows)

        def run_slice(sl, init_ref, out_ref, ebase):
            def init_i(i, c):
                r = pl.multiple_of(row0 + i * STG, 8)
                pltpu.sync_copy(init_ref.at[pl.ds(r, STG), :], stage)
                pltpu.sync_copy(stage, acc.at[pl.ds(r, STG), :])
                return c
            lax.fori_loop(0, n_wb, init_i, 0)
            plsc.subcore_barrier()

            def blk(b, c):
                eoff = pl.multiple_of((ebase + b * BLK_E) // RPD, 8)
                pltpu.sync_copy(src2.at[pl.ds(eoff, DMAS), :], sidx)
                pltpu.sync_copy(dst2.at[pl.ds(eoff, DMAS), :], didx)
                if not ones_mode:
                    descs = [pltpu.async_copy(gs[sl].at[sidx.at[j]], rows.at[j],
                                              sem_g) for j in range(DMAS)]
                    for d in descs:
                        d.wait()
                for j in range(DMAS):
                    pltpu.sync_copy(rows.at[j], acc.at[didx.at[j]], add=True)
                return c
            lax.fori_loop(0, nblk, blk, 0)
            plsc.subcore_barrier()

            def wb_i(i, c):
                r = pl.multiple_of(row0 + i * STG, 8)
                pltpu.sync_copy(acc.at[pl.ds(r, STG), :], stage)
                pltpu.sync_copy(stage, out_ref.at[pl.ds(r, STG), :])
                return c
            lax.fori_loop(0, n_wb, wb_i, 0)
            plsc.subcore_barrier()

        if edge_split:
            ebase = (cid * NS + sid) * epr_t

            @pl.when(cid == 0)
            def _():
                run_slice(0, gs[0], outs[0], ebase)

            @pl.when(cid == 1)
            def _():
                run_slice(0, zinit, outs[1], ebase)
        else:
            for sl in range(nsl):
                @pl.when(cid == (sl % NC))
                def _(sl=sl):
                    run_slice(sl, gs[sl], outs[sl], sid * epr_t)

    out_type = [jax.ShapeDtypeStruct((n, SLW), F32)] * nouts
    scratch = [
        pltpu.VMEM((DMAS, RPD), jnp.int32),
        pltpu.VMEM((DMAS, RPD), jnp.int32),
        pltpu.VMEM((DMAS, RPD, SLW), F32),
        pltpu.VMEM((STG, SLW), F32),
        pltpu.VMEM_SHARED((n, SLW), F32),
        pltpu.SemaphoreType.DMA,
    ]
    return pl.kernel(body, out_type=out_type, mesh=_mesh(),
                     scratch_types=scratch,
                     compiler_params=pltpu.CompilerParams(
                         use_tc_tiling_on_sc=False))


_NBLK = 2048
_PBLK = _NBLK // 16


def _full(shape):
    return pl.BlockSpec(shape, lambda i: tuple(0 for _ in shape))


def _rows(w):
    return pl.BlockSpec((_NBLK, w), lambda i: (i, 0))


def _prow():
    return pl.BlockSpec((_PBLK, 128), lambda i: (i, 0))


def _pout(n):
    return jax.ShapeDtypeStruct((n // 16, 128), F32)


def _tc_pre0(xP, p0P, p1P, W8):
    n16 = xP.shape[0]

    def body(x_r, p0_r, p1_r, w_r, disP_r, *g_r):
        disP = lax.rsqrt(p0_r[...] + p1_r[...])
        disP_r[...] = disP
        gk = []
        for k in range(16):
            xk = x_r[:, 8 * k:8 * k + 8]
            d = disP[:, 8 * k:8 * k + 1]
            gk.append(d * jnp.dot(xk, w_r[...], preferred_element_type=F32))
        for s in range(4):
            g_r[s][...] = jnp.concatenate(
                [gk[k][:, 8 * s:8 * s + 8] for k in range(16)], axis=1)

    outs = [jax.ShapeDtypeStruct((n16, 128), F32)] * 5
    return pl.pallas_call(
        body, grid=(n16 // _PBLK,),
        in_specs=[_prow(), _prow(), _prow(), _full((SLW, 32))],
        out_specs=[_prow()] * 5,
        out_shape=outs)(xP, p0P, p1P, W8)


def _tc_fuse(s_parts, disP, bP, Wn, nsl_out, relu=True):
    nin = len(s_parts)
    wi, wo = Wn.shape
    n16 = disP.shape[0]

    def body(*refs):
        s_r = refs[:nin]
        disP_r = refs[nin]
        b_r = refs[nin + 1:nin + 1 + nin]
        w_r = refs[nin + 1 + nin]
        g_r = refs[nin + 2 + nin:]
        disP = disP_r[...]
        aggs = []
        for s in range(nin):
            a = disP * s_r[s][...] + b_r[s][...]
            if relu:
                a = jnp.maximum(a, 0.0)
            aggs.append(a)
        Gk = []
        for k in range(16):
            Ak = jnp.concatenate([aggs[s][:, 8 * k:8 * k + 8]
                                  for s in range(nin)], axis=1)
            Rk = jnp.dot(Ak, w_r[...], preferred_element_type=F32)
            Gk.append(disP[:, 8 * k:8 * k + 1] * Rk)
        for s2 in range(nsl_out):
            g_r[s2][...] = jnp.concatenate(
                [Gk[k][:, 8 * s2:8 * s2 + 8] for k in range(16)], axis=1)

    return pl.pallas_call(
        body, grid=(n16 // _PBLK,),
        in_specs=([_prow()] * (nin + 1) + [_full((1, 128))] * nin +
                  [_full((wi, wo))]),
        out_specs=[_prow()] * nsl_out,
        out_shape=[jax.ShapeDtypeStruct((n16, 128), F32)] * nsl_out,
    )(*s_parts, disP, *bP, Wn)


def _tc_post(s_parts, p0P, p1P, b, wout, relu):
    n16 = p0P.shape[0]
    n = n16 * 16
    nin = len(s_parts)

    def body(*refs):
        s_r = refs[:nin]
        p0_r, p1_r, b_r, o_r = refs[nin:nin + 4]
        disP = lax.rsqrt(p0_r[...] + p1_r[...])
        for k in range(16):
            sg = jnp.concatenate([s_r[s][:, 8 * k:8 * k + 8]
                                  for s in range(nin)], axis=1)
            out = disP[:, 8 * k:8 * k + 1] * sg + b_r[...]
            if relu:
                out = jnp.maximum(out, 0.0)
            o_r[pl.Slice(k, 128, 16), :] = out[:, :wout]

    wcat = nin * SLW
    return pl.pallas_call(
        body, grid=(n16 // _PBLK,),
        in_specs=[_prow()] * (nin + 2) + [_full((1, wcat))],
        out_specs=_rows(wout),
        out_shape=jax.ShapeDtypeStruct((n, wout), F32),
    )(*s_parts, p0P, p1P, b)


def _tc_post_sum2(q0P, q1P, p0P, p1P, b, wout):
    n16 = p0P.shape[0]
    n = n16 * 16

    def body(q0_r, q1_r, p0_r, p1_r, b_r, o_r):
        disP = lax.rsqrt(p0_r[...] + p1_r[...])
        sgP = q0_r[...] + q1_r[...]
        for k in range(16):
            out = disP[:, 8 * k:8 * k + 1] * sgP[:, 8 * k:8 * k + 8] + b_r[...]
            o_r[pl.Slice(k, 128, 16), :] = out[:, :wout]

    return pl.pallas_call(
        body, grid=(n16 // _PBLK,),
        in_specs=[_prow()] * 4 + [_full((1, SLW))],
        out_specs=_rows(wout),
        out_shape=jax.ShapeDtypeStruct((n, wout), F32),
    )(q0P, q1P, p0P, p1P, b)


def _tc_pre_dec(dmat, p0P, p1P, Wgd8):
    n = dmat.shape[0]

    def body(d_r, p0_r, p1_r, w_r, g_r):
        disP = lax.rsqrt(p0_r[...] + p1_r[...])
        gk = []
        for k in range(16):
            dk = d_r[pl.Slice(k, 128, 16), :]
            gk.append(disP[:, 8 * k:8 * k + 1] *
                      jnp.dot(dk, w_r[...], preferred_element_type=F32))
        g_r[...] = jnp.concatenate(gk, axis=1)

    return pl.pallas_call(
        body, grid=(n // _NBLK,),
        in_specs=[_rows(32), _prow(), _prow(), _full((32, SLW))],
        out_specs=_prow(),
        out_shape=jax.ShapeDtypeStruct((n // 16, 128), F32),
    )(dmat, p0P, p1P, Wgd8)


_MBLK = 1024


def _tc_mlp(hf, eps, enc_w, enc_b, Wmu, bmu, Wlv, blv, dec_w, dec_b, Wout, bout):
    m, fin = hf.shape
    lat = Wmu.shape[1]
    ne, nd = len(enc_w), len(dec_w)

    def body(*refs):
        i = 0
        hf_r = refs[i]; i += 1
        eps_r = refs[i]; i += 1
        ew = refs[i:i + ne]; i += ne
        eb = refs[i:i + ne]; i += ne
        wmu_r, bmu_r, wlv_r, blv_r = refs[i:i + 4]; i += 4
        dw = refs[i:i + nd]; i += nd
        db = refs[i:i + nd]; i += nd
        wout_r, bout_r = refs[i:i + 2]; i += 2
        mu_r, lv_r, z_r, d_r = refs[i:i + 4]

        h = hf_r[...]
        for j in range(ne):
            h = jnp.maximum(jnp.dot(h, ew[j][...],
                                    preferred_element_type=F32) + eb[j][...], 0.0)
        mu = jnp.dot(h, wmu_r[...], preferred_element_type=F32) + bmu_r[...]
        lv = jnp.dot(h, wlv_r[...], preferred_element_type=F32) + blv_r[...]
        z = mu + eps_r[...] * jnp.exp(0.5 * lv)
        mu_r[...] = mu
        lv_r[...] = lv
        z_r[...] = z
        d = z
        for j in range(nd):
            d = jnp.maximum(jnp.dot(d, dw[j][...],
                                    preferred_element_type=F32) + db[j][...], 0.0)
        d_r[...] = jnp.dot(d, wout_r[...], preferred_element_type=F32) + bout_r[...]

    row = lambda w: pl.BlockSpec((_MBLK, w), lambda i: (i, 0))
    in_specs = ([row(fin), row(lat)] +
                [_full(w.shape) for w in enc_w] + [_full(b.shape) for b in enc_b] +
                [_full(Wmu.shape), _full(bmu.shape), _full(Wlv.shape), _full(blv.shape)] +
                [_full(w.shape) for w in dec_w] + [_full(b.shape) for b in dec_b] +
                [_full(Wout.shape), _full(bout.shape)])
    outs = [jax.ShapeDtypeStruct((m, lat), F32)] * 3 + \
           [jax.ShapeDtypeStruct((m, fin), F32)]
    return pl.pallas_call(
        body, grid=(m // _MBLK,),
        in_specs=in_specs,
        out_specs=[row(lat)] * 3 + [row(fin)],
        out_shape=outs,
    )(hf, eps, *enc_w, *enc_b, Wmu, bmu, Wlv, blv, *dec_w, *dec_b, Wout, bout)


def kernel(x, edge_index, params):
    n = x.shape[0]
    e = edge_index.shape[1]
    src2 = edge_index[0].reshape(-1, RPD)
    dst2 = edge_index[1].reshape(-1, RPD)

    sc = lambda a: a.reshape(n, SLW)
    tc = lambda a: a.reshape(n // 16, 128)
    bP = lambda b, nsl: [jnp.tile(b[SLW * s:SLW * (s + 1)], 16).reshape(1, 128)
                         for s in range(nsl)]

    ones_n = jnp.ones((n, SLW), F32)
    zeros_n = jnp.zeros((n, SLW), F32)
    ones_blk = jnp.ones((DMAS, RPD, SLW), F32)

    deg_k = _make_spmm(n, e, 1, edge_split=True, ones_mode=True)
    p0, p1 = deg_k(src2, dst2, ones_n, zeros_n, ones_blk)

    xP = tc(jnp.pad(x, ((0, 0), (0, SLW - 3))))
    W8 = jnp.pad(params['Wg0'], ((0, SLW - 3), (0, 0)))
    disP, g00, g01, g02, g03 = _tc_pre0(xP, tc(p0), tc(p1), W8)

    spmm4 = _make_spmm(n, e, 4, edge_split=False)
    spmm8 = _make_spmm(n, e, 8, edge_split=False)
    spmm1 = _make_spmm(n, e, 1, edge_split=True)

    s0 = spmm4(src2, dst2, sc(g00), sc(g01), sc(g02), sc(g03))
    g1 = _tc_fuse([tc(a) for a in s0], disP, bP(params['bg0'], 4),
                  params['Wg1'], 8)
    s1 = spmm8(src2, dst2, *[sc(a) for a in g1])
    g2 = _tc_fuse([tc(a) for a in s1], disP, bP(params['bg1'], 8),
                  params['Wg2'], 4)
    s2 = spmm4(src2, dst2, *[sc(a) for a in g2])
    hmat = _tc_post([tc(a) for a in s2], tc(p0), tc(p1),
                    params['bg2'].reshape(1, -1), 32, relu=False)

    hf = hmat.reshape(-1, 32 * 22)
    eps = jax.random.normal(jax.random.key(42), (hf.shape[0], 56), F32)
    enc_w = [params['We%d' % i] for i in range(5)]
    enc_b = [params['be%d' % i].reshape(1, -1) for i in range(5)]
    dec_w = [params['Wd%d' % i] for i in range(5)]
    dec_b = [params['bd%d' % i].reshape(1, -1) for i in range(5)]
    mu, logvar, z, dflat = _tc_mlp(
        hf, eps, enc_w, enc_b,
        params['Wmu'], params['bmu'].reshape(1, -1),
        params['Wlv'], params['blv'].reshape(1, -1),
        dec_w, dec_b, params['Wout'], params['bout'].reshape(1, -1))

    dmat = dflat.reshape(-1, 32)
    Wgd8 = jnp.pad(params['Wgd'], ((0, 0), (0, SLW - 3)))
    bgd8 = jnp.pad(params['bgd'], (0, SLW - 3)).reshape(1, -1)
    g3 = _tc_pre_dec(dmat, tc(p0), tc(p1), Wgd8)
    q0, q1 = spmm1(src2, dst2, sc(g3), zeros_n)
    x_rec = _tc_post_sum2(tc(q0), tc(q1), tc(p0), tc(p1), bgd8, 3)

    return (x_rec, z, mu, logvar)


# speedup vs baseline: 25.7015x; 1.0378x over previous
import functools

import jax
import jax.numpy as jnp
from jax import lax
from jax.experimental import pallas as pl
from jax.experimental.pallas import tpu as pltpu
from jax.experimental.pallas import tpu_sc as plsc

F32 = jnp.float32
NC, NS = 2, 16
RPD = 128
DMAS = 16
BLK_E = RPD * DMAS
SLW = 8
STG = 1408


def _mesh():
    return plsc.VectorSubcoreMesh(core_axis_name="c", subcore_axis_name="s",
                                  num_cores=NC, num_subcores=NS)


def _make_spmm(n, e, nsl, edge_split, ones_mode=False):
    rows_per_tile = n // NS
    n_wb = rows_per_tile // STG
    nouts = 2 if edge_split else nsl
    epr_t = e // (NC * NS) if edge_split else e // NS
    nblk = epr_t // BLK_E
    assert rows_per_tile % STG == 0 and epr_t % BLK_E == 0

    n_in = 2 + nsl + (1 if edge_split else 0) + (1 if ones_mode else 0)

    def body(*refs):
        src2, dst2 = refs[0], refs[1]
        gs = refs[2:2 + nsl]
        k = 2 + nsl
        zinit = refs[k] if edge_split else None
        k += 1 if edge_split else 0
        ones_blk = refs[k] if ones_mode else None
        k += 1 if ones_mode else 0
        outs = refs[k:k + nouts]
        sidx, didx, rows, stage, acc, sem_g = refs[k + nouts:]
        cid = lax.axis_index("c")
        sid = lax.axis_index("s")
        row0 = pl.multiple_of(sid * rows_per_tile, 8)

        if ones_mode:
            pltpu.sync_copy(ones_blk, rows)

        def run_slice(sl, init_ref, out_ref, ebase):
            def init_i(i, c):
                r = pl.multiple_of(row0 + i * STG, 8)
                pltpu.sync_copy(init_ref.at[pl.ds(r, STG), :], stage)
                pltpu.sync_copy(stage, acc.at[pl.ds(r, STG), :])
                return c
            lax.fori_loop(0, n_wb, init_i, 0)
            plsc.subcore_barrier()

            def blk(b, c):
                eoff = pl.multiple_of((ebase + b * BLK_E) // RPD, 8)
                pltpu.sync_copy(src2.at[pl.ds(eoff, DMAS), :], sidx)
                pltpu.sync_copy(dst2.at[pl.ds(eoff, DMAS), :], didx)
                if not ones_mode:
                    descs = [pltpu.async_copy(gs[sl].at[sidx.at[j]], rows.at[j],
                                              sem_g) for j in range(DMAS)]
                    for d in descs:
                        d.wait()
                for j in range(DMAS):
                    pltpu.sync_copy(rows.at[j], acc.at[didx.at[j]], add=True)
                return c
            lax.fori_loop(0, nblk, blk, 0)
            plsc.subcore_barrier()

            def wb_i(i, c):
                r = pl.multiple_of(row0 + i * STG, 8)
                pltpu.sync_copy(acc.at[pl.ds(r, STG), :], stage)
                pltpu.sync_copy(stage, out_ref.at[pl.ds(r, STG), :])
                return c
            lax.fori_loop(0, n_wb, wb_i, 0)
            plsc.subcore_barrier()

        if edge_split:
            ebase = (cid * NS + sid) * epr_t

            @pl.when(cid == 0)
            def _():
                run_slice(0, gs[0], outs[0], ebase)

            @pl.when(cid == 1)
            def _():
                run_slice(0, zinit, outs[1], ebase)
        else:
            for sl in range(nsl):
                @pl.when(cid == (sl % NC))
                def _(sl=sl):
                    run_slice(sl, gs[sl], outs[sl], sid * epr_t)

    out_type = [jax.ShapeDtypeStruct((n, SLW), F32)] * nouts
    scratch = [
        pltpu.VMEM((DMAS, RPD), jnp.int32),
        pltpu.VMEM((DMAS, RPD), jnp.int32),
        pltpu.VMEM((DMAS, RPD, SLW), F32),
        pltpu.VMEM((STG, SLW), F32),
        pltpu.VMEM_SHARED((n, SLW), F32),
        pltpu.SemaphoreType.DMA,
    ]
    return pl.kernel(body, out_type=out_type, mesh=_mesh(),
                     scratch_types=scratch,
                     compiler_params=pltpu.CompilerParams(
                         use_tc_tiling_on_sc=False))


_NBLK = 2048
_PBLK = _NBLK // 16


def _full(shape):
    return pl.BlockSpec(shape, lambda i: tuple(0 for _ in shape))


def _rows(w):
    return pl.BlockSpec((_NBLK, w), lambda i: (i, 0))


def _prow():
    return pl.BlockSpec((_PBLK, 128), lambda i: (i, 0))


def _pout(n):
    return jax.ShapeDtypeStruct((n // 16, 128), F32)


def _tc_pre0(x, p0P, p1P, W):
    n = x.shape[0]
    n16 = n // 16

    def body(x_r, p0_r, p1_r, w_r, disP_r, *g_r):
        disP = lax.rsqrt(p0_r[...] + p1_r[...])
        disP_r[...] = disP
        gk = []
        for k in range(16):
            xk = x_r[pl.Slice(k, 128, 16), :]
            d = disP[:, 8 * k:8 * k + 1]
            gk.append(d * jnp.dot(xk, w_r[...], preferred_element_type=F32))
        for s in range(4):
            g_r[s][...] = jnp.concatenate(
                [gk[k][:, 8 * s:8 * s + 8] for k in range(16)], axis=1)

    outs = [jax.ShapeDtypeStruct((n16, 128), F32)] * 5
    return pl.pallas_call(
        body, grid=(n16 // _PBLK,),
        in_specs=[_rows(3), _prow(), _prow(), _full((3, 32))],
        out_specs=[_prow()] * 5,
        out_shape=outs)(x, p0P, p1P, W)


def _tc_fuse(s_parts, disP, bP, Wn, nsl_out, relu=True):
    nin = len(s_parts)
    wi, wo = Wn.shape
    n16 = disP.shape[0]

    def body(*refs):
        s_r = refs[:nin]
        disP_r = refs[nin]
        b_r = refs[nin + 1:nin + 1 + nin]
        w_r = refs[nin + 1 + nin]
        g_r = refs[nin + 2 + nin:]
        disP = disP_r[...]
        aggs = []
        for s in range(nin):
            a = disP * s_r[s][...] + b_r[s][...]
            if relu:
                a = jnp.maximum(a, 0.0)
            aggs.append(a)
        Gk = []
        for k in range(16):
            Ak = jnp.concatenate([aggs[s][:, 8 * k:8 * k + 8]
                                  for s in range(nin)], axis=1)
            Rk = jnp.dot(Ak, w_r[...], preferred_element_type=F32)
            Gk.append(disP[:, 8 * k:8 * k + 1] * Rk)
        for s2 in range(nsl_out):
            g_r[s2][...] = jnp.concatenate(
                [Gk[k][:, 8 * s2:8 * s2 + 8] for k in range(16)], axis=1)

    return pl.pallas_call(
        body, grid=(n16 // _PBLK,),
        in_specs=([_prow()] * (nin + 1) + [_full((1, 128))] * nin +
                  [_full((wi, wo))]),
        out_specs=[_prow()] * nsl_out,
        out_shape=[jax.ShapeDtypeStruct((n16, 128), F32)] * nsl_out,
    )(*s_parts, disP, *bP, Wn)


def _tc_post(s_parts, p0P, p1P, b, wout, relu):
    n16 = p0P.shape[0]
    n = n16 * 16
    nin = len(s_parts)

    def body(*refs):
        s_r = refs[:nin]
        p0_r, p1_r, b_r, o_r = refs[nin:nin + 4]
        disP = lax.rsqrt(p0_r[...] + p1_r[...])
        for k in range(16):
            sg = jnp.concatenate([s_r[s][:, 8 * k:8 * k + 8]
                                  for s in range(nin)], axis=1)
            out = disP[:, 8 * k:8 * k + 1] * sg + b_r[...]
            if relu:
                out = jnp.maximum(out, 0.0)
            o_r[pl.Slice(k, 128, 16), :] = out[:, :wout]

    wcat = nin * SLW
    return pl.pallas_call(
        body, grid=(n16 // _PBLK,),
        in_specs=[_prow()] * (nin + 2) + [_full((1, wcat))],
        out_specs=_rows(wout),
        out_shape=jax.ShapeDtypeStruct((n, wout), F32),
    )(*s_parts, p0P, p1P, b)


def _tc_post_sum2(q0P, q1P, p0P, p1P, b, wout):
    n16 = p0P.shape[0]
    n = n16 * 16

    def body(q0_r, q1_r, p0_r, p1_r, b_r, o_r):
        disP = lax.rsqrt(p0_r[...] + p1_r[...])
        sgP = q0_r[...] + q1_r[...]
        for k in range(16):
            out = disP[:, 8 * k:8 * k + 1] * sgP[:, 8 * k:8 * k + 8] + b_r[...]
            o_r[pl.Slice(k, 128, 16), :] = out[:, :wout]

    return pl.pallas_call(
        body, grid=(n16 // _PBLK,),
        in_specs=[_prow()] * 4 + [_full((1, SLW))],
        out_specs=_rows(wout),
        out_shape=jax.ShapeDtypeStruct((n, wout), F32),
    )(q0P, q1P, p0P, p1P, b)


def _tc_pre_dec(dmat, p0P, p1P, Wgd8):
    n = dmat.shape[0]

    def body(d_r, p0_r, p1_r, w_r, g_r):
        disP = lax.rsqrt(p0_r[...] + p1_r[...])
        gk = []
        for k in range(16):
            dk = d_r[pl.Slice(k, 128, 16), :]
            gk.append(disP[:, 8 * k:8 * k + 1] *
                      jnp.dot(dk, w_r[...], preferred_element_type=F32))
        g_r[...] = jnp.concatenate(gk, axis=1)

    return pl.pallas_call(
        body, grid=(n // _NBLK,),
        in_specs=[_rows(32), _prow(), _prow(), _full((32, SLW))],
        out_specs=_prow(),
        out_shape=jax.ShapeDtypeStruct((n // 16, 128), F32),
    )(dmat, p0P, p1P, Wgd8)


_MBLK = 1024


def _tc_mlp(hf, eps, enc_w, enc_b, Wmu, bmu, Wlv, blv, dec_w, dec_b, Wout, bout):
    m, fin = hf.shape
    lat = Wmu.shape[1]
    ne, nd = len(enc_w), len(dec_w)

    def body(*refs):
        i = 0
        hf_r = refs[i]; i += 1
        eps_r = refs[i]; i += 1
        ew = refs[i:i + ne]; i += ne
        eb = refs[i:i + ne]; i += ne
        wmu_r, bmu_r, wlv_r, blv_r = refs[i:i + 4]; i += 4
        dw = refs[i:i + nd]; i += nd
        db = refs[i:i + nd]; i += nd
        wout_r, bout_r = refs[i:i + 2]; i += 2
        mu_r, lv_r, z_r, d_r = refs[i:i + 4]

        h = hf_r[...]
        for j in range(ne):
            h = jnp.maximum(jnp.dot(h, ew[j][...],
                                    preferred_element_type=F32) + eb[j][...], 0.0)
        mu = jnp.dot(h, wmu_r[...], preferred_element_type=F32) + bmu_r[...]
        lv = jnp.dot(h, wlv_r[...], preferred_element_type=F32) + blv_r[...]
        z = mu + eps_r[...] * jnp.exp(0.5 * lv)
        mu_r[...] = mu
        lv_r[...] = lv
        z_r[...] = z
        d = z
        for j in range(nd):
            d = jnp.maximum(jnp.dot(d, dw[j][...],
                                    preferred_element_type=F32) + db[j][...], 0.0)
        d_r[...] = jnp.dot(d, wout_r[...], preferred_element_type=F32) + bout_r[...]

    row = lambda w: pl.BlockSpec((_MBLK, w), lambda i: (i, 0))
    in_specs = ([row(fin), row(lat)] +
                [_full(w.shape) for w in enc_w] + [_full(b.shape) for b in enc_b] +
                [_full(Wmu.shape), _full(bmu.shape), _full(Wlv.shape), _full(blv.shape)] +
                [_full(w.shape) for w in dec_w] + [_full(b.shape) for b in dec_b] +
                [_full(Wout.shape), _full(bout.shape)])
    outs = [jax.ShapeDtypeStruct((m, lat), F32)] * 3 + \
           [jax.ShapeDtypeStruct((m, fin), F32)]
    return pl.pallas_call(
        body, grid=(m // _MBLK,),
        in_specs=in_specs,
        out_specs=[row(lat)] * 3 + [row(fin)],
        out_shape=outs,
    )(hf, eps, *enc_w, *enc_b, Wmu, bmu, Wlv, blv, *dec_w, *dec_b, Wout, bout)


def kernel(x, edge_index, params):
    n = x.shape[0]
    e = edge_index.shape[1]
    src2 = edge_index[0].reshape(-1, RPD)
    dst2 = edge_index[1].reshape(-1, RPD)

    sc = lambda a: a.reshape(n, SLW)
    tc = lambda a: a.reshape(n // 16, 128)
    bP = lambda b, nsl: [jnp.tile(b[SLW * s:SLW * (s + 1)], 16).reshape(1, 128)
                         for s in range(nsl)]

    ones_n = jnp.ones((n, SLW), F32)
    zeros_n = jnp.zeros((n, SLW), F32)
    ones_blk = jnp.ones((DMAS, RPD, SLW), F32)

    deg_k = _make_spmm(n, e, 1, edge_split=True, ones_mode=True)
    p0, p1 = deg_k(src2, dst2, ones_n, zeros_n, ones_blk)

    disP, g00, g01, g02, g03 = _tc_pre0(x, tc(p0), tc(p1), params['Wg0'])

    spmm4 = _make_spmm(n, e, 4, edge_split=False)
    spmm8 = _make_spmm(n, e, 8, edge_split=False)
    spmm1 = _make_spmm(n, e, 1, edge_split=True)

    s0 = spmm4(src2, dst2, sc(g00), sc(g01), sc(g02), sc(g03))
    g1 = _tc_fuse([tc(a) for a in s0], disP, bP(params['bg0'], 4),
                  params['Wg1'], 8)
    s1 = spmm8(src2, dst2, *[sc(a) for a in g1])
    g2 = _tc_fuse([tc(a) for a in s1], disP, bP(params['bg1'], 8),
                  params['Wg2'], 4)
    s2 = spmm4(src2, dst2, *[sc(a) for a in g2])
    hmat = _tc_post([tc(a) for a in s2], tc(p0), tc(p1),
                    params['bg2'].reshape(1, -1), 32, relu=False)

    hf = hmat.reshape(-1, 32 * 22)
    eps = jax.random.normal(jax.random.key(42), (hf.shape[0], 56), F32)
    enc_w = [params['We%d' % i] for i in range(5)]
    enc_b = [params['be%d' % i].reshape(1, -1) for i in range(5)]
    dec_w = [params['Wd%d' % i] for i in range(5)]
    dec_b = [params['bd%d' % i].reshape(1, -1) for i in range(5)]
    mu, logvar, z, dflat = _tc_mlp(
        hf, eps, enc_w, enc_b,
        params['Wmu'], params['bmu'].reshape(1, -1),
        params['Wlv'], params['blv'].reshape(1, -1),
        dec_w, dec_b, params['Wout'], params['bout'].reshape(1, -1))

    dmat = dflat.reshape(-1, 32)
    Wgd8 = jnp.pad(params['Wgd'], ((0, 0), (0, SLW - 3)))
    bgd8 = jnp.pad(params['bgd'], (0, SLW - 3)).reshape(1, -1)
    g3 = _tc_pre_dec(dmat, tc(p0), tc(p1), Wgd8)
    q0, q1 = spmm1(src2, dst2, sc(g3), zeros_n)
    x_rec = _tc_post_sum2(tc(q0), tc(q1), tc(p0), tc(p1), bgd8, 3)

    return (x_rec, z, mu, logvar)
